# baseline (speedup 1.0000x reference)
import jax
import jax.numpy as jnp
import numpy as np
from jax.experimental import pallas as pl
from jax.experimental.pallas import tpu as pltpu

_K = 8192
_D = 256
_TEMP_INV = 2.0
_LOG_K = float(np.log(_K))
_R = 128

_HIGHEST = jax.lax.Precision.HIGHEST
_NT_DIMS = (((1,), (1,)), ((), ()))


def _vq_body(z_ref, cb_ref, g_ref, zq_ref, hard_ref, kl_ref, cm_ref, cc_ref):
    i = pl.program_id(0)
    cb = cb_ref[...]

    @pl.when(i == 0)
    def _init():
        kl_ref[...] = jnp.zeros_like(kl_ref)
        cm_ref[...] = jnp.zeros_like(cm_ref)
        ones = jnp.ones((1, _D), jnp.float32)
        cc_ref[...] = jax.lax.dot_general(
            ones, cb * cb, _NT_DIMS, precision=_HIGHEST,
            preferred_element_type=jnp.float32)

    z = z_ref[...]
    g = g_ref[...]
    zz = jnp.sum(z * z, axis=1, keepdims=True)
    cross = jax.lax.dot_general(
        z, cb, _NT_DIMS, precision=jax.lax.Precision.DEFAULT,
        preferred_element_type=jnp.float32)
    dist = (cc_ref[...] + zz) - 2.0 * cross

    y = (g - dist) * _TEMP_INV
    m = jnp.max(y, axis=1, keepdims=True)
    e = jnp.exp(y - m)
    s = jnp.sum(e, axis=1, keepdims=True)
    inv_s = 1.0 / s
    hard_ref[...] = jnp.argmax(y, axis=1).astype(jnp.int32)[:, None]
    zq = jax.lax.dot_general(
        e, cb, (((1,), (0,)), ((), ())), precision=jax.lax.Precision.DEFAULT,
        preferred_element_type=jnp.float32)
    zq_ref[...] = zq * inv_s

    m2 = jnp.min(dist, axis=1, keepdims=True)
    e2 = jnp.exp(m2 - dist)
    s2 = jnp.sum(e2, axis=1, keepdims=True)
    inv_s2 = 1.0 / s2
    row_cm = jnp.sum(e2 * dist, axis=1, keepdims=True) * inv_s2
    row_kl = (m2 + (_LOG_K - jnp.log(s2))) - row_cm
    kl_ref[...] += jnp.sum(row_kl, keepdims=True)
    cm_ref[...] += jnp.sum(row_cm, keepdims=True)


def kernel(z_e, codebook, gumbel):
    bs, feat, w, h = z_e.shape
    n = bs * w * h
    z = jnp.transpose(z_e, (0, 2, 3, 1)).reshape(n, feat)
    grid = (n // _R,)
    zq, hard, kl, cm = pl.pallas_call(
        _vq_body,
        grid=grid,
        in_specs=[
            pl.BlockSpec((_R, _D), lambda i: (i, 0)),
            pl.BlockSpec((_K, _D), lambda i: (0, 0)),
            pl.BlockSpec((_R, _K), lambda i: (i, 0)),
        ],
        out_specs=[
            pl.BlockSpec((_R, _D), lambda i: (i, 0)),
            pl.BlockSpec((_R, 1), lambda i: (i, 0)),
            pl.BlockSpec((1, 1), lambda i: (0, 0)),
            pl.BlockSpec((1, 1), lambda i: (0, 0)),
        ],
        out_shape=[
            jax.ShapeDtypeStruct((n, _D), jnp.float32),
            jax.ShapeDtypeStruct((n, 1), jnp.int32),
            jax.ShapeDtypeStruct((1, 1), jnp.float32),
            jax.ShapeDtypeStruct((1, 1), jnp.float32),
        ],
        scratch_shapes=[pltpu.VMEM((1, _K), jnp.float32)],
    )(z, codebook, gumbel)
    z_q = jnp.transpose(zq.reshape(bs, w, h, feat), (0, 3, 1, 2))
    hard_indices = hard.reshape(bs, w, h)
    inv_bs = np.float32(1.0 / bs)
    return (z_q, hard_indices, kl[0, 0] * inv_bs, cm[0, 0] * inv_bs)


# speedup vs baseline: 1.0192x; 1.0192x over previous
import jax
import jax.numpy as jnp
import numpy as np
from jax.experimental import pallas as pl
from jax.experimental.pallas import tpu as pltpu

_K = 8192
_D = 256
_TEMP_INV = 2.0
_LOG_K = float(np.log(_K))
_R = 128

_HIGHEST = jax.lax.Precision.HIGHEST
_NT_DIMS = (((1,), (1,)), ((), ()))


def _vq_body(z_ref, cb_ref, cb16_ref, g_ref, zq_ref, hard_ref, kl_ref, cm_ref,
             cc_ref):
    i = pl.program_id(0)
    cb16 = cb16_ref[...]

    @pl.when(i == 0)
    def _init():
        cb = cb_ref[...]
        kl_ref[...] = jnp.zeros_like(kl_ref)
        cm_ref[...] = jnp.zeros_like(cm_ref)
        ones = jnp.ones((1, _D), jnp.float32)
        cc_ref[...] = jax.lax.dot_general(
            ones, cb * cb, _NT_DIMS, precision=_HIGHEST,
            preferred_element_type=jnp.float32)

    z = z_ref[...]
    g = g_ref[...]
    zz = jnp.sum(z * z, axis=1, keepdims=True)
    cross = jax.lax.dot_general(
        z.astype(jnp.bfloat16), cb16, _NT_DIMS,
        precision=jax.lax.Precision.DEFAULT,
        preferred_element_type=jnp.float32)
    dist = (cc_ref[...] + zz) - 2.0 * cross

    y = (g - dist) * _TEMP_INV
    m = jnp.max(y, axis=1, keepdims=True)
    e = jnp.exp(y - m)
    s = jnp.sum(e, axis=1, keepdims=True)
    inv_s = 1.0 / s
    hard_ref[...] = jnp.argmax(y, axis=1).astype(jnp.int32)[:, None]
    zq = jax.lax.dot_general(
        e.astype(jnp.bfloat16), cb16, (((1,), (0,)), ((), ())),
        precision=jax.lax.Precision.DEFAULT,
        preferred_element_type=jnp.float32)
    zq_ref[...] = zq * inv_s

    m2 = jnp.min(dist, axis=1, keepdims=True)
    e2 = jnp.exp(m2 - dist)
    s2 = jnp.sum(e2, axis=1, keepdims=True)
    inv_s2 = 1.0 / s2
    row_cm = jnp.sum(e2 * dist, axis=1, keepdims=True) * inv_s2
    row_kl = (m2 + (_LOG_K - jnp.log(s2))) - row_cm
    kl_ref[...] += jnp.sum(row_kl, keepdims=True)
    cm_ref[...] += jnp.sum(row_cm, keepdims=True)


def kernel(z_e, codebook, gumbel):
    bs, feat, w, h = z_e.shape
    n = bs * w * h
    z = jnp.transpose(z_e, (0, 2, 3, 1)).reshape(n, feat)
    grid = (n // _R,)
    zq, hard, kl, cm = pl.pallas_call(
        _vq_body,
        grid=grid,
        in_specs=[
            pl.BlockSpec((_R, _D), lambda i: (i, 0)),
            pl.BlockSpec((_K, _D), lambda i: (0, 0)),
            pl.BlockSpec((_K, _D), lambda i: (0, 0)),
            pl.BlockSpec((_R, _K), lambda i: (i, 0)),
        ],
        out_specs=[
            pl.BlockSpec((_R, _D), lambda i: (i, 0)),
            pl.BlockSpec((_R, 1), lambda i: (i, 0)),
            pl.BlockSpec((1, 1), lambda i: (0, 0)),
            pl.BlockSpec((1, 1), lambda i: (0, 0)),
        ],
        out_shape=[
            jax.ShapeDtypeStruct((n, _D), jnp.float32),
            jax.ShapeDtypeStruct((n, 1), jnp.int32),
            jax.ShapeDtypeStruct((1, 1), jnp.float32),
            jax.ShapeDtypeStruct((1, 1), jnp.float32),
        ],
        scratch_shapes=[pltpu.VMEM((1, _K), jnp.float32)],
    )(z, codebook, codebook.astype(jnp.bfloat16), gumbel)
    z_q = jnp.transpose(zq.reshape(bs, w, h, feat), (0, 3, 1, 2))
    hard_indices = hard.reshape(bs, w, h)
    inv_bs = np.float32(1.0 / bs)
    return (z_q, hard_indices, kl[0, 0] * inv_bs, cm[0, 0] * inv_bs)
